# Initial kernel scaffold; baseline (speedup 1.0000x reference)
import jax
import jax.numpy as jnp
from jax.experimental import pallas as pl


def kernel(edge_index, n_users, n_items, user_emb, item_emb):
    raise NotImplementedError("write your pallas kernel here")


# speedup vs baseline: 33.5210x; 33.5210x over previous
import jax
import jax.numpy as jnp
from jax import lax
from jax.experimental import pallas as pl
from jax.experimental.pallas import tpu as pltpu
from jax.experimental.pallas import tpu_sc as plsc

_N = 50000
_NP = 50176
_DH = 16
_G = 128
_GPT = 784
_EP = 16 * _GPT * _G
_CG = 8
_NCHUNK = _GPT // _CG
_RPT = _NP // 16
_RC = 224
_NRC = _RPT // _RC
_NLAYERS = 3


def _rsqrt_newton(d):
    bits = lax.bitcast_convert_type(d, jnp.int32)
    seed = jnp.int32(0x5F3759DF) - lax.shift_right_logical(bits, 1)
    y = lax.bitcast_convert_type(seed, jnp.float32)
    for _ in range(3):
        y = y * (1.5 - 0.5 * d * y * y)
    return jnp.where(d >= 0.5, y, jnp.float32(0.0))


def _body(srcg, dstg, x0h, zrow, zdeg, out,
          y_sh, acc_sh, deg_sh,
          srcv, dstv, rows, xbuf, obuf, abuf, ones_v,
          gsem, ssem):
    c = lax.axis_index("c")
    s = lax.axis_index("s")
    row0 = s * _RPT
    g0 = s * _GPT

    def _fill_ones(i, carry):
        ones_v[pl.ds(i * 16, 16)] = jnp.ones((16,), jnp.float32)
        return carry
    lax.fori_loop(0, _G // 16, _fill_ones, 0)

    def _zero_acc(k, carry):
        pltpu.sync_copy(zrow, acc_sh.at[pl.ds(row0 + k * _RC, _RC)])
        return carry

    pltpu.sync_copy(zdeg, deg_sh.at[pl.ds(row0, _RPT)])
    lax.fori_loop(0, _NRC, _zero_acc, 0)
    plsc.subcore_barrier()

    def _deg_chunk(i, carry):
        pltpu.sync_copy(dstg.at[pl.ds(g0 + i * _CG, _CG)], dstv)
        descs = [pltpu.async_copy(ones_v, deg_sh.at[dstv.at[j]], ssem,
                                  add=True) for j in range(_CG)]
        for d in descs:
            d.wait()
        return carry
    lax.fori_loop(0, _NCHUNK, _deg_chunk, 0)
    plsc.subcore_barrier()

    def _prologue_chunk(k, carry):
        r0 = row0 + k * _RC

        pltpu.sync_copy(x0h.at[c, pl.ds(r0, _RC)], xbuf)
        pltpu.sync_copy(xbuf, out.at[c, pl.ds(r0, _RC)])
        pltpu.sync_copy(deg_sh.at[pl.ds(r0, _RC)], abuf)

        def _scale0(i, c2):
            avec = _rsqrt_newton(abuf[pl.ds(i * 16, 16)])
            for t in range(16):
                r = i * 16 + t
                xbuf[r] = xbuf[r] * avec[t]
            return c2
        lax.fori_loop(0, _RC // 16, _scale0, 0)
        pltpu.sync_copy(xbuf, y_sh.at[pl.ds(r0, _RC)])
        return carry
    lax.fori_loop(0, _NRC, _prologue_chunk, 0)
    plsc.subcore_barrier()

    for layer in range(_NLAYERS):
        def _edge_chunk(i, carry):
            gg = g0 + i * _CG
            pltpu.sync_copy(srcg.at[pl.ds(gg, _CG)], srcv)
            pltpu.sync_copy(dstg.at[pl.ds(gg, _CG)], dstv)
            descs = [pltpu.async_copy(y_sh.at[srcv.at[j]], rows.at[j], gsem)
                     for j in range(_CG)]
            for d in descs:
                d.wait()
            descs = [pltpu.async_copy(rows.at[j], acc_sh.at[dstv.at[j]], ssem,
                                      add=True) for j in range(_CG)]
            for d in descs:
                d.wait()
            return carry
        lax.fori_loop(0, _NCHUNK, _edge_chunk, 0)
        plsc.subcore_barrier()

        last = layer == _NLAYERS - 1
        if not last:
            def _resc_chunk(k, carry):
                r0 = row0 + k * _RC
                pltpu.sync_copy(acc_sh.at[pl.ds(r0, _RC)], xbuf)
                pltpu.sync_copy(out.at[c, pl.ds(r0, _RC)], obuf)
                pltpu.sync_copy(deg_sh.at[pl.ds(r0, _RC)], abuf)

                def _resc(i, c2):
                    avec = _rsqrt_newton(abuf[pl.ds(i * 16, 16)])
                    for t in range(16):
                        r = i * 16 + t
                        xv = xbuf[r] * avec[t]
                        obuf[r] = obuf[r] + xv
                        xbuf[r] = xv * avec[t]
                    return c2
                lax.fori_loop(0, _RC // 16, _resc, 0)
                pltpu.sync_copy(obuf, out.at[c, pl.ds(r0, _RC)])
                pltpu.sync_copy(xbuf, y_sh.at[pl.ds(r0, _RC)])
                return carry
            lax.fori_loop(0, _NRC, _resc_chunk, 0)
            lax.fori_loop(0, _NRC, _zero_acc, 0)
        else:
            def _resc_chunk(k, carry):
                r0 = row0 + k * _RC
                pltpu.sync_copy(acc_sh.at[pl.ds(r0, _RC)], xbuf)
                pltpu.sync_copy(out.at[c, pl.ds(r0, _RC)], obuf)
                pltpu.sync_copy(deg_sh.at[pl.ds(r0, _RC)], abuf)

                def _resc(i, c2):
                    avec = _rsqrt_newton(abuf[pl.ds(i * 16, 16)])
                    for t in range(16):
                        r = i * 16 + t
                        xv = xbuf[r] * avec[t]
                        obuf[r] = (obuf[r] + xv) * jnp.float32(0.25)
                    return c2
                lax.fori_loop(0, _RC // 16, _resc, 0)
                pltpu.sync_copy(obuf, out.at[c, pl.ds(r0, _RC)])
                return carry
            lax.fori_loop(0, _NRC, _resc_chunk, 0)
        plsc.subcore_barrier()


def _propagate(srcp, dstp, x0h, zrow, zdeg):
    mesh = plsc.VectorSubcoreMesh(core_axis_name="c", subcore_axis_name="s")
    f = pl.kernel(
        _body,
        out_type=jax.ShapeDtypeStruct((2, _NP, _DH), jnp.float32),
        mesh=mesh,
        scratch_types=[
            pltpu.VMEM_SHARED((_NP, _DH), jnp.float32),
            pltpu.VMEM_SHARED((_NP, _DH), jnp.float32),
            pltpu.VMEM_SHARED((_NP,), jnp.float32),
            pltpu.VMEM((_CG, _G), jnp.int32),
            pltpu.VMEM((_CG, _G), jnp.int32),
            pltpu.VMEM((_CG, _G, _DH), jnp.float32),
            pltpu.VMEM((_RC, _DH), jnp.float32),
            pltpu.VMEM((_RC, _DH), jnp.float32),
            pltpu.VMEM((_RC,), jnp.float32),
            pltpu.VMEM((_G,), jnp.float32),
            pltpu.SemaphoreType.DMA,
            pltpu.SemaphoreType.DMA,
        ],
        compiler_params=pltpu.CompilerParams(use_tc_tiling_on_sc=False),
    )
    return f(srcp, dstp, x0h, zrow, zdeg)


def kernel(edge_index, n_users, n_items, user_emb, item_emb):
    nu = user_emb.shape[0]
    ni = item_emb.shape[0]
    n = nu + ni
    x0 = jnp.concatenate([user_emb, item_emb], axis=0)
    x0h = jnp.stack([x0[:, :_DH], x0[:, _DH:]])
    x0h = jnp.concatenate(
        [x0h, jnp.zeros((2, _NP - n, _DH), x0.dtype)], axis=1)
    pad = _EP - edge_index.shape[1]
    srcp = jnp.concatenate(
        [edge_index[0], jnp.full((pad,), n, jnp.int32)]).reshape(-1, _G)
    dstp = jnp.concatenate(
        [edge_index[1], jnp.full((pad,), n, jnp.int32)]).reshape(-1, _G)
    zrow = jnp.zeros((_RC, _DH), jnp.float32)
    zdeg = jnp.zeros((_RPT,), jnp.float32)
    out = _propagate(srcp, dstp, x0h, zrow, zdeg)
    x = jnp.concatenate([out[0, :n], out[1, :n]], axis=1)
    return (x[:nu], x[nu:])


# speedup vs baseline: 36.7210x; 1.0955x over previous
import jax
import jax.numpy as jnp
from jax import lax
from jax.experimental import pallas as pl
from jax.experimental.pallas import tpu as pltpu
from jax.experimental.pallas import tpu_sc as plsc

_N = 50000
_NP = 50176
_DH = 16
_G = 128
_GPT = 784
_EP = 16 * _GPT * _G
_CG = 8
_NCHUNK = _GPT // _CG
_RPT = _NP // 16
_RC = 112
_NRC = _RPT // _RC
_NLAYERS = 3


def _rsqrt_newton(d):
    bits = lax.bitcast_convert_type(d, jnp.int32)
    seed = jnp.int32(0x5F3759DF) - lax.shift_right_logical(bits, 1)
    y = lax.bitcast_convert_type(seed, jnp.float32)
    for _ in range(3):
        y = y * (1.5 - 0.5 * d * y * y)
    return jnp.where(d >= 0.5, y, jnp.float32(0.0))


def _body(srcg, dstg, x0h, zrow, zdeg, out,
          y_sh, acc_sh, deg_sh,
          srcv0, dstv0, srcv1, dstv1, rows, xbuf, obuf, abuf, ones_v,
          gsem, ssem, isem0, isem1):
    c = lax.axis_index("c")
    s = lax.axis_index("s")
    row0 = s * _RPT
    g0 = s * _GPT

    def _prime_idx(gg):
        pltpu.async_copy(srcg.at[pl.ds(gg, _CG)], srcv0, isem0)
        pltpu.async_copy(dstg.at[pl.ds(gg, _CG)], dstv0, isem0)

    def _wait_idx(gg, sv, dv, isem):
        pltpu.make_async_copy(srcg.at[pl.ds(gg, _CG)], sv, isem).wait()
        pltpu.make_async_copy(dstg.at[pl.ds(gg, _CG)], dv, isem).wait()

    def _prefetch_idx(gg, sv, dv, isem):
        pltpu.async_copy(srcg.at[pl.ds(gg, _CG)], sv, isem)
        pltpu.async_copy(dstg.at[pl.ds(gg, _CG)], dv, isem)

    def _fill_ones(i, carry):
        ones_v[pl.ds(i * 16, 16)] = jnp.ones((16,), jnp.float32)
        return carry
    lax.fori_loop(0, _G // 16, _fill_ones, 0)

    def _zero_acc(k, carry):
        pltpu.sync_copy(zrow, acc_sh.at[pl.ds(row0 + k * _RC, _RC)])
        return carry

    pltpu.sync_copy(zdeg, deg_sh.at[pl.ds(row0, _RPT)])
    lax.fori_loop(0, _NRC, _zero_acc, 0)
    plsc.subcore_barrier()

    def _deg_phase(i, dv, isem, nsv, ndv, nisem, is_b):
        gg = g0 + (2 * i + (1 if is_b else 0)) * _CG
        pltpu.make_async_copy(dstg.at[pl.ds(gg, _CG)], dv, isem).wait()
        nxt = gg + _CG

        @pl.when(nxt < g0 + _GPT)
        def _():
            pltpu.async_copy(dstg.at[pl.ds(nxt, _CG)], ndv, nisem)
        descs = [pltpu.async_copy(ones_v, deg_sh.at[dv.at[j]], ssem,
                                  add=True) for j in range(_CG)]
        for d in descs:
            d.wait()

    pltpu.async_copy(dstg.at[pl.ds(g0, _CG)], dstv0, isem0)

    def _deg_pair(i, carry):
        _deg_phase(i, dstv0, isem0, None, dstv1, isem1, False)
        _deg_phase(i, dstv1, isem1, None, dstv0, isem0, True)
        return carry
    lax.fori_loop(0, _NCHUNK // 2, _deg_pair, 0)
    plsc.subcore_barrier()

    def _prologue_chunk(k, carry):
        r0 = row0 + k * _RC

        pltpu.sync_copy(x0h.at[c, pl.ds(r0, _RC)], xbuf)
        pltpu.sync_copy(xbuf, out.at[c, pl.ds(r0, _RC)])
        pltpu.sync_copy(deg_sh.at[pl.ds(r0, _RC)], abuf)

        def _scale0(i, c2):
            avec = _rsqrt_newton(abuf[pl.ds(i * 16, 16)])
            for t in range(16):
                r = i * 16 + t
                xbuf[r] = xbuf[r] * avec[t]
            return c2
        lax.fori_loop(0, _RC // 16, _scale0, 0)
        pltpu.sync_copy(xbuf, y_sh.at[pl.ds(r0, _RC)])
        return carry
    lax.fori_loop(0, _NRC, _prologue_chunk, 0)
    plsc.subcore_barrier()

    def _edge_phase(i, sv, dv, isem, nsv, ndv, nisem, is_b):
        gg = g0 + (2 * i + (1 if is_b else 0)) * _CG
        _wait_idx(gg, sv, dv, isem)
        nxt = gg + _CG

        @pl.when(nxt < g0 + _GPT)
        def _():
            _prefetch_idx(nxt, nsv, ndv, nisem)
        gd = [pltpu.async_copy(y_sh.at[sv.at[j]], rows.at[j], gsem)
              for j in range(_CG)]
        sd = []
        for j in range(_CG):
            gd[j].wait()
            sd.append(pltpu.async_copy(rows.at[j], acc_sh.at[dv.at[j]], ssem,
                                       add=True))
        for d in sd:
            d.wait()

    for layer in range(_NLAYERS):
        _prime_idx(g0)

        def _edge_pair(i, carry):
            _edge_phase(i, srcv0, dstv0, isem0, srcv1, dstv1, isem1, False)
            _edge_phase(i, srcv1, dstv1, isem1, srcv0, dstv0, isem0, True)
            return carry
        lax.fori_loop(0, _NCHUNK // 2, _edge_pair, 0)
        plsc.subcore_barrier()

        last = layer == _NLAYERS - 1
        if not last:
            def _resc_chunk(k, carry):
                r0 = row0 + k * _RC
                pltpu.sync_copy(acc_sh.at[pl.ds(r0, _RC)], xbuf)
                pltpu.sync_copy(out.at[c, pl.ds(r0, _RC)], obuf)
                pltpu.sync_copy(deg_sh.at[pl.ds(r0, _RC)], abuf)

                def _resc(i, c2):
                    avec = _rsqrt_newton(abuf[pl.ds(i * 16, 16)])
                    for t in range(16):
                        r = i * 16 + t
                        xv = xbuf[r] * avec[t]
                        obuf[r] = obuf[r] + xv
                        xbuf[r] = xv * avec[t]
                    return c2
                lax.fori_loop(0, _RC // 16, _resc, 0)
                pltpu.sync_copy(obuf, out.at[c, pl.ds(r0, _RC)])
                pltpu.sync_copy(xbuf, y_sh.at[pl.ds(r0, _RC)])
                return carry
            lax.fori_loop(0, _NRC, _resc_chunk, 0)
            lax.fori_loop(0, _NRC, _zero_acc, 0)
        else:
            def _resc_chunk(k, carry):
                r0 = row0 + k * _RC
                pltpu.sync_copy(acc_sh.at[pl.ds(r0, _RC)], xbuf)
                pltpu.sync_copy(out.at[c, pl.ds(r0, _RC)], obuf)
                pltpu.sync_copy(deg_sh.at[pl.ds(r0, _RC)], abuf)

                def _resc(i, c2):
                    avec = _rsqrt_newton(abuf[pl.ds(i * 16, 16)])
                    for t in range(16):
                        r = i * 16 + t
                        xv = xbuf[r] * avec[t]
                        obuf[r] = (obuf[r] + xv) * jnp.float32(0.25)
                    return c2
                lax.fori_loop(0, _RC // 16, _resc, 0)
                pltpu.sync_copy(obuf, out.at[c, pl.ds(r0, _RC)])
                return carry
            lax.fori_loop(0, _NRC, _resc_chunk, 0)
        plsc.subcore_barrier()


def _propagate(srcp, dstp, x0h, zrow, zdeg):
    mesh = plsc.VectorSubcoreMesh(core_axis_name="c", subcore_axis_name="s")
    f = pl.kernel(
        _body,
        out_type=jax.ShapeDtypeStruct((2, _NP, _DH), jnp.float32),
        mesh=mesh,
        scratch_types=[
            pltpu.VMEM_SHARED((_NP, _DH), jnp.float32),
            pltpu.VMEM_SHARED((_NP, _DH), jnp.float32),
            pltpu.VMEM_SHARED((_NP,), jnp.float32),
            pltpu.VMEM((_CG, _G), jnp.int32),
            pltpu.VMEM((_CG, _G), jnp.int32),
            pltpu.VMEM((_CG, _G), jnp.int32),
            pltpu.VMEM((_CG, _G), jnp.int32),
            pltpu.VMEM((_CG, _G, _DH), jnp.float32),
            pltpu.VMEM((_RC, _DH), jnp.float32),
            pltpu.VMEM((_RC, _DH), jnp.float32),
            pltpu.VMEM((_RC,), jnp.float32),
            pltpu.VMEM((_G,), jnp.float32),
            pltpu.SemaphoreType.DMA,
            pltpu.SemaphoreType.DMA,
            pltpu.SemaphoreType.DMA,
            pltpu.SemaphoreType.DMA,
        ],
        compiler_params=pltpu.CompilerParams(use_tc_tiling_on_sc=False),
    )
    return f(srcp, dstp, x0h, zrow, zdeg)


def kernel(edge_index, n_users, n_items, user_emb, item_emb):
    nu = user_emb.shape[0]
    ni = item_emb.shape[0]
    n = nu + ni
    x0 = jnp.concatenate([user_emb, item_emb], axis=0)
    x0h = jnp.stack([x0[:, :_DH], x0[:, _DH:]])
    x0h = jnp.concatenate(
        [x0h, jnp.zeros((2, _NP - n, _DH), x0.dtype)], axis=1)
    pad = _EP - edge_index.shape[1]
    srcp = jnp.concatenate(
        [edge_index[0], jnp.full((pad,), n, jnp.int32)]).reshape(-1, _G)
    dstp = jnp.concatenate(
        [edge_index[1], jnp.full((pad,), n, jnp.int32)]).reshape(-1, _G)
    zrow = jnp.zeros((_RC, _DH), jnp.float32)
    zdeg = jnp.zeros((_RPT,), jnp.float32)
    out = _propagate(srcp, dstp, x0h, zrow, zdeg)
    x = jnp.concatenate([out[0, :n], out[1, :n]], axis=1)
    return (x[:nu], x[nu:])


# speedup vs baseline: 39.1353x; 1.0657x over previous
import jax
import jax.numpy as jnp
from jax import lax
from jax.experimental import pallas as pl
from jax.experimental.pallas import tpu as pltpu
from jax.experimental.pallas import tpu_sc as plsc

_N = 50000
_NP = 50176
_DH = 16
_G = 128
_GPT = 784
_EP = 16 * _GPT * _G
_CG = 8
_NCHUNK = _GPT // _CG
_RPT = _NP // 16
_RC = 112
_NRC = _RPT // _RC
_NLAYERS = 3


def _rsqrt_newton(d):
    bits = lax.bitcast_convert_type(d, jnp.int32)
    seed = jnp.int32(0x5F3759DF) - lax.shift_right_logical(bits, 1)
    y = lax.bitcast_convert_type(seed, jnp.float32)
    for _ in range(3):
        y = y * (1.5 - 0.5 * d * y * y)
    return jnp.where(d >= 0.5, y, jnp.float32(0.0))


def _body(srcg, dstg, x0p, zrow, zdeg, out,
          y_sh, acc_sh, deg_sh,
          srcv0, dstv0, srcv1, dstv1, rows, xbuf, obuf, abuf, ones_v,
          gsem, ssem, isem0, isem1):
    c = lax.axis_index("c")
    s = lax.axis_index("s")
    row0 = s * _RPT
    g0 = s * _GPT
    dcol = c * _DH

    def _prime_idx(gg):
        pltpu.async_copy(srcg.at[pl.ds(gg, _CG)], srcv0, isem0)
        pltpu.async_copy(dstg.at[pl.ds(gg, _CG)], dstv0, isem0)

    def _wait_idx(gg, sv, dv, isem):
        pltpu.make_async_copy(srcg.at[pl.ds(gg, _CG)], sv, isem).wait()
        pltpu.make_async_copy(dstg.at[pl.ds(gg, _CG)], dv, isem).wait()

    def _prefetch_idx(gg, sv, dv, isem):
        pltpu.async_copy(srcg.at[pl.ds(gg, _CG)], sv, isem)
        pltpu.async_copy(dstg.at[pl.ds(gg, _CG)], dv, isem)

    def _fill_ones(i, carry):
        ones_v[pl.ds(i * 16, 16)] = jnp.ones((16,), jnp.float32)
        return carry
    lax.fori_loop(0, _G // 16, _fill_ones, 0)

    def _zero_acc(k, carry):
        pltpu.sync_copy(zrow, acc_sh.at[pl.ds(row0 + k * _RC, _RC)])
        return carry

    pltpu.sync_copy(zdeg, deg_sh.at[pl.ds(row0, _RPT)])
    lax.fori_loop(0, _NRC, _zero_acc, 0)
    plsc.subcore_barrier()

    def _deg_phase(i, dv, isem, nsv, ndv, nisem, is_b):
        gg = g0 + (2 * i + (1 if is_b else 0)) * _CG
        pltpu.make_async_copy(dstg.at[pl.ds(gg, _CG)], dv, isem).wait()
        nxt = gg + _CG

        @pl.when(nxt < g0 + _GPT)
        def _():
            pltpu.async_copy(dstg.at[pl.ds(nxt, _CG)], ndv, nisem)
        descs = [pltpu.async_copy(ones_v, deg_sh.at[dv.at[j]], ssem,
                                  add=True) for j in range(_CG)]
        for d in descs:
            d.wait()

    pltpu.async_copy(dstg.at[pl.ds(g0, _CG)], dstv0, isem0)

    def _deg_pair(i, carry):
        _deg_phase(i, dstv0, isem0, None, dstv1, isem1, False)
        _deg_phase(i, dstv1, isem1, None, dstv0, isem0, True)
        return carry
    lax.fori_loop(0, _NCHUNK // 2, _deg_pair, 0)
    plsc.subcore_barrier()

    def _prologue_chunk(k, carry):
        r0 = row0 + k * _RC

        pltpu.sync_copy(x0p.at[pl.ds(r0, _RC), pl.ds(dcol, _DH)], xbuf)
        pltpu.sync_copy(xbuf, out.at[pl.ds(r0, _RC), pl.ds(dcol, _DH)])
        pltpu.sync_copy(deg_sh.at[pl.ds(r0, _RC)], abuf)

        def _scale0(i, c2):
            avec = _rsqrt_newton(abuf[pl.ds(i * 16, 16)])
            for t in range(16):
                r = i * 16 + t
                xbuf[r] = xbuf[r] * avec[t]
            return c2
        lax.fori_loop(0, _RC // 16, _scale0, 0)
        pltpu.sync_copy(xbuf, y_sh.at[pl.ds(r0, _RC)])
        return carry
    lax.fori_loop(0, _NRC, _prologue_chunk, 0)
    plsc.subcore_barrier()

    def _edge_phase(i, sv, dv, isem, nsv, ndv, nisem, is_b):
        gg = g0 + (2 * i + (1 if is_b else 0)) * _CG
        _wait_idx(gg, sv, dv, isem)
        nxt = gg + _CG

        @pl.when(nxt < g0 + _GPT)
        def _():
            _prefetch_idx(nxt, nsv, ndv, nisem)
        gd = [pltpu.async_copy(y_sh.at[sv.at[j]], rows.at[j], gsem)
              for j in range(_CG)]
        sd = []
        for j in range(_CG):
            gd[j].wait()
            sd.append(pltpu.async_copy(rows.at[j], acc_sh.at[dv.at[j]], ssem,
                                       add=True))
        for d in sd:
            d.wait()

    for layer in range(_NLAYERS):
        _prime_idx(g0)

        def _edge_pair(i, carry):
            _edge_phase(i, srcv0, dstv0, isem0, srcv1, dstv1, isem1, False)
            _edge_phase(i, srcv1, dstv1, isem1, srcv0, dstv0, isem0, True)
            return carry
        lax.fori_loop(0, _NCHUNK // 2, _edge_pair, 0)
        plsc.subcore_barrier()

        last = layer == _NLAYERS - 1
        if not last:
            def _resc_chunk(k, carry):
                r0 = row0 + k * _RC
                pltpu.sync_copy(acc_sh.at[pl.ds(r0, _RC)], xbuf)
                pltpu.sync_copy(out.at[pl.ds(r0, _RC), pl.ds(dcol, _DH)], obuf)
                pltpu.sync_copy(deg_sh.at[pl.ds(r0, _RC)], abuf)

                def _resc(i, c2):
                    avec = _rsqrt_newton(abuf[pl.ds(i * 16, 16)])
                    for t in range(16):
                        r = i * 16 + t
                        xv = xbuf[r] * avec[t]
                        obuf[r] = obuf[r] + xv
                        xbuf[r] = xv * avec[t]
                    return c2
                lax.fori_loop(0, _RC // 16, _resc, 0)
                pltpu.sync_copy(obuf, out.at[pl.ds(r0, _RC), pl.ds(dcol, _DH)])
                pltpu.sync_copy(xbuf, y_sh.at[pl.ds(r0, _RC)])
                return carry
            lax.fori_loop(0, _NRC, _resc_chunk, 0)
            lax.fori_loop(0, _NRC, _zero_acc, 0)
        else:
            def _resc_chunk(k, carry):
                r0 = row0 + k * _RC
                pltpu.sync_copy(acc_sh.at[pl.ds(r0, _RC)], xbuf)
                pltpu.sync_copy(out.at[pl.ds(r0, _RC), pl.ds(dcol, _DH)], obuf)
                pltpu.sync_copy(deg_sh.at[pl.ds(r0, _RC)], abuf)

                def _resc(i, c2):
                    avec = _rsqrt_newton(abuf[pl.ds(i * 16, 16)])
                    for t in range(16):
                        r = i * 16 + t
                        xv = xbuf[r] * avec[t]
                        obuf[r] = (obuf[r] + xv) * jnp.float32(0.25)
                    return c2
                lax.fori_loop(0, _RC // 16, _resc, 0)
                pltpu.sync_copy(obuf, out.at[pl.ds(r0, _RC), pl.ds(dcol, _DH)])
                return carry
            lax.fori_loop(0, _NRC, _resc_chunk, 0)
        plsc.subcore_barrier()


def _propagate(srcp, dstp, x0p, zrow, zdeg):
    mesh = plsc.VectorSubcoreMesh(core_axis_name="c", subcore_axis_name="s")
    f = pl.kernel(
        _body,
        out_type=jax.ShapeDtypeStruct((_NP, 2 * _DH), jnp.float32),
        mesh=mesh,
        scratch_types=[
            pltpu.VMEM_SHARED((_NP, _DH), jnp.float32),
            pltpu.VMEM_SHARED((_NP, _DH), jnp.float32),
            pltpu.VMEM_SHARED((_NP,), jnp.float32),
            pltpu.VMEM((_CG, _G), jnp.int32),
            pltpu.VMEM((_CG, _G), jnp.int32),
            pltpu.VMEM((_CG, _G), jnp.int32),
            pltpu.VMEM((_CG, _G), jnp.int32),
            pltpu.VMEM((_CG, _G, _DH), jnp.float32),
            pltpu.VMEM((_RC, _DH), jnp.float32),
            pltpu.VMEM((_RC, _DH), jnp.float32),
            pltpu.VMEM((_RC,), jnp.float32),
            pltpu.VMEM((_G,), jnp.float32),
            pltpu.SemaphoreType.DMA,
            pltpu.SemaphoreType.DMA,
            pltpu.SemaphoreType.DMA,
            pltpu.SemaphoreType.DMA,
        ],
        compiler_params=pltpu.CompilerParams(use_tc_tiling_on_sc=False),
    )
    return f(srcp, dstp, x0p, zrow, zdeg)


def kernel(edge_index, n_users, n_items, user_emb, item_emb):
    nu = user_emb.shape[0]
    ni = item_emb.shape[0]
    n = nu + ni
    x0p = jnp.concatenate(
        [user_emb, item_emb, jnp.zeros((_NP - n, 2 * _DH), user_emb.dtype)],
        axis=0)
    pad = _EP - edge_index.shape[1]
    srcp = jnp.concatenate(
        [edge_index[0], jnp.full((pad,), n, jnp.int32)]).reshape(-1, _G)
    dstp = jnp.concatenate(
        [edge_index[1], jnp.full((pad,), n, jnp.int32)]).reshape(-1, _G)
    zrow = jnp.zeros((_RC, _DH), jnp.float32)
    zdeg = jnp.zeros((_RPT,), jnp.float32)
    out = _propagate(srcp, dstp, x0p, zrow, zdeg)
    return (out[:nu], out[nu:n])


# speedup vs baseline: 39.6584x; 1.0134x over previous
import jax
import jax.numpy as jnp
from jax import lax
from jax.experimental import pallas as pl
from jax.experimental.pallas import tpu as pltpu
from jax.experimental.pallas import tpu_sc as plsc

_N = 50000
_NP = 50176
_DH = 16
_G = 128
_GPT = 784
_EP = 16 * _GPT * _G
_CG = 4
_NCHUNK = _GPT // _CG
_RPT = _NP // 16
_RC = 112
_NRC = _RPT // _RC
_NLAYERS = 3


def _rsqrt_newton(d):
    bits = lax.bitcast_convert_type(d, jnp.int32)
    seed = jnp.int32(0x5F3759DF) - lax.shift_right_logical(bits, 1)
    y = lax.bitcast_convert_type(seed, jnp.float32)
    for _ in range(3):
        y = y * (1.5 - 0.5 * d * y * y)
    return jnp.where(d >= 0.5, y, jnp.float32(0.0))


def _body(srcg, dstg, x0p, zrow, zdeg, out,
          y_sh, acc_sh, deg_sh,
          srcv0, dstv0, srcv1, dstv1, rows0, rows1, xbuf, obuf, abuf, ones_v,
          gsem, ssem0, ssem1, isem0, isem1):
    c = lax.axis_index("c")
    s = lax.axis_index("s")
    row0 = s * _RPT
    g0 = s * _GPT
    dcol = c * _DH

    def _prime_idx(gg):
        pltpu.async_copy(srcg.at[pl.ds(gg, _CG)], srcv0, isem0)
        pltpu.async_copy(dstg.at[pl.ds(gg, _CG)], dstv0, isem0)

    def _wait_idx(gg, sv, dv, isem):
        pltpu.make_async_copy(srcg.at[pl.ds(gg, _CG)], sv, isem).wait()
        pltpu.make_async_copy(dstg.at[pl.ds(gg, _CG)], dv, isem).wait()

    def _prefetch_idx(gg, sv, dv, isem):
        pltpu.async_copy(srcg.at[pl.ds(gg, _CG)], sv, isem)
        pltpu.async_copy(dstg.at[pl.ds(gg, _CG)], dv, isem)

    def _fill_ones(i, carry):
        ones_v[pl.ds(i * 16, 16)] = jnp.ones((16,), jnp.float32)
        return carry
    lax.fori_loop(0, _G // 16, _fill_ones, 0)

    def _zero_acc(k, carry):
        pltpu.sync_copy(zrow, acc_sh.at[pl.ds(row0 + k * _RC, _RC)])
        return carry

    pltpu.sync_copy(zdeg, deg_sh.at[pl.ds(row0, _RPT)])
    lax.fori_loop(0, _NRC, _zero_acc, 0)
    plsc.subcore_barrier()

    def _deg_drain(dv, ssem):
        for j in range(_CG):
            pltpu.make_async_copy(ones_v, deg_sh.at[dv.at[j]], ssem).wait()

    def _deg_phase(i, off, dv, isem, ssem, odv, oisem, ossem):
        gg = g0 + (2 * i + off) * _CG
        if off == 1:
            _deg_drain(odv, ossem)
        else:
            @pl.when(i > 0)
            def _():
                _deg_drain(odv, ossem)
        if off == 0:
            pltpu.async_copy(dstg.at[pl.ds(gg + _CG, _CG)], odv, oisem)
        else:
            @pl.when(i < _NCHUNK // 2 - 1)
            def _():
                pltpu.async_copy(dstg.at[pl.ds(gg + _CG, _CG)], odv, oisem)
        pltpu.make_async_copy(dstg.at[pl.ds(gg, _CG)], dv, isem).wait()
        for j in range(_CG):
            pltpu.async_copy(ones_v, deg_sh.at[dv.at[j]], ssem, add=True)

    pltpu.async_copy(dstg.at[pl.ds(g0, _CG)], dstv0, isem0)

    def _deg_pair(i, carry):
        _deg_phase(i, 0, dstv0, isem0, ssem0, dstv1, isem1, ssem1)
        _deg_phase(i, 1, dstv1, isem1, ssem1, dstv0, isem0, ssem0)
        return carry
    lax.fori_loop(0, _NCHUNK // 2, _deg_pair, 0)
    _deg_drain(dstv1, ssem1)
    plsc.subcore_barrier()

    def _prologue_chunk(k, carry):
        r0 = row0 + k * _RC

        pltpu.sync_copy(x0p.at[pl.ds(r0, _RC), pl.ds(dcol, _DH)], xbuf)
        pltpu.sync_copy(xbuf, out.at[pl.ds(r0, _RC), pl.ds(dcol, _DH)])
        pltpu.sync_copy(deg_sh.at[pl.ds(r0, _RC)], abuf)

        def _scale0(i, c2):
            avec = _rsqrt_newton(abuf[pl.ds(i * 16, 16)])
            for t in range(16):
                r = i * 16 + t
                xbuf[r] = xbuf[r] * avec[t]
            return c2
        lax.fori_loop(0, _RC // 16, _scale0, 0)
        pltpu.sync_copy(xbuf, y_sh.at[pl.ds(r0, _RC)])
        return carry
    lax.fori_loop(0, _NRC, _prologue_chunk, 0)
    plsc.subcore_barrier()

    def _edge_drain(rws, dv, ssem):
        for j in range(_CG):
            pltpu.make_async_copy(rws.at[j], acc_sh.at[dv.at[j]], ssem).wait()

    def _edge_phase(i, off, sv, dv, isem, ssem, rws,
                    osv, odv, oisem, ossem, orws):
        gg = g0 + (2 * i + off) * _CG
        if off == 1:
            _edge_drain(orws, odv, ossem)
        else:
            @pl.when(i > 0)
            def _():
                _edge_drain(orws, odv, ossem)
        if off == 0:
            _prefetch_idx(gg + _CG, osv, odv, oisem)
        else:
            @pl.when(i < _NCHUNK // 2 - 1)
            def _():
                _prefetch_idx(gg + _CG, osv, odv, oisem)
        _wait_idx(gg, sv, dv, isem)
        gd = [pltpu.async_copy(y_sh.at[sv.at[j]], rws.at[j], gsem)
              for j in range(_CG)]
        for j in range(_CG):
            gd[j].wait()
            pltpu.async_copy(rws.at[j], acc_sh.at[dv.at[j]], ssem, add=True)

    for layer in range(_NLAYERS):
        _prime_idx(g0)

        def _edge_pair(i, carry):
            _edge_phase(i, 0, srcv0, dstv0, isem0, ssem0, rows0,
                        srcv1, dstv1, isem1, ssem1, rows1)
            _edge_phase(i, 1, srcv1, dstv1, isem1, ssem1, rows1,
                        srcv0, dstv0, isem0, ssem0, rows0)
            return carry
        lax.fori_loop(0, _NCHUNK // 2, _edge_pair, 0)
        _edge_drain(rows1, dstv1, ssem1)
        plsc.subcore_barrier()

        last = layer == _NLAYERS - 1
        if not last:
            def _resc_chunk(k, carry):
                r0 = row0 + k * _RC
                pltpu.sync_copy(acc_sh.at[pl.ds(r0, _RC)], xbuf)
                pltpu.sync_copy(out.at[pl.ds(r0, _RC), pl.ds(dcol, _DH)], obuf)
                pltpu.sync_copy(deg_sh.at[pl.ds(r0, _RC)], abuf)

                def _resc(i, c2):
                    avec = _rsqrt_newton(abuf[pl.ds(i * 16, 16)])
                    for t in range(16):
                        r = i * 16 + t
                        xv = xbuf[r] * avec[t]
                        obuf[r] = obuf[r] + xv
                        xbuf[r] = xv * avec[t]
                    return c2
                lax.fori_loop(0, _RC // 16, _resc, 0)
                pltpu.sync_copy(obuf, out.at[pl.ds(r0, _RC), pl.ds(dcol, _DH)])
                pltpu.sync_copy(xbuf, y_sh.at[pl.ds(r0, _RC)])
                return carry
            lax.fori_loop(0, _NRC, _resc_chunk, 0)
            lax.fori_loop(0, _NRC, _zero_acc, 0)
        else:
            def _resc_chunk(k, carry):
                r0 = row0 + k * _RC
                pltpu.sync_copy(acc_sh.at[pl.ds(r0, _RC)], xbuf)
                pltpu.sync_copy(out.at[pl.ds(r0, _RC), pl.ds(dcol, _DH)], obuf)
                pltpu.sync_copy(deg_sh.at[pl.ds(r0, _RC)], abuf)

                def _resc(i, c2):
                    avec = _rsqrt_newton(abuf[pl.ds(i * 16, 16)])
                    for t in range(16):
                        r = i * 16 + t
                        xv = xbuf[r] * avec[t]
                        obuf[r] = (obuf[r] + xv) * jnp.float32(0.25)
                    return c2
                lax.fori_loop(0, _RC // 16, _resc, 0)
                pltpu.sync_copy(obuf, out.at[pl.ds(r0, _RC), pl.ds(dcol, _DH)])
                return carry
            lax.fori_loop(0, _NRC, _resc_chunk, 0)
        plsc.subcore_barrier()


def _propagate(srcp, dstp, x0p, zrow, zdeg):
    mesh = plsc.VectorSubcoreMesh(core_axis_name="c", subcore_axis_name="s")
    f = pl.kernel(
        _body,
        out_type=jax.ShapeDtypeStruct((_NP, 2 * _DH), jnp.float32),
        mesh=mesh,
        scratch_types=[
            pltpu.VMEM_SHARED((_NP, _DH), jnp.float32),
            pltpu.VMEM_SHARED((_NP, _DH), jnp.float32),
            pltpu.VMEM_SHARED((_NP,), jnp.float32),
            pltpu.VMEM((_CG, _G), jnp.int32),
            pltpu.VMEM((_CG, _G), jnp.int32),
            pltpu.VMEM((_CG, _G), jnp.int32),
            pltpu.VMEM((_CG, _G), jnp.int32),
            pltpu.VMEM((_CG, _G, _DH), jnp.float32),
            pltpu.VMEM((_CG, _G, _DH), jnp.float32),
            pltpu.VMEM((_RC, _DH), jnp.float32),
            pltpu.VMEM((_RC, _DH), jnp.float32),
            pltpu.VMEM((_RC,), jnp.float32),
            pltpu.VMEM((_G,), jnp.float32),
            pltpu.SemaphoreType.DMA,
            pltpu.SemaphoreType.DMA,
            pltpu.SemaphoreType.DMA,
            pltpu.SemaphoreType.DMA,
            pltpu.SemaphoreType.DMA,
        ],
        compiler_params=pltpu.CompilerParams(use_tc_tiling_on_sc=False),
    )
    return f(srcp, dstp, x0p, zrow, zdeg)


def kernel(edge_index, n_users, n_items, user_emb, item_emb):
    nu = user_emb.shape[0]
    ni = item_emb.shape[0]
    n = nu + ni
    x0p = jnp.concatenate(
        [user_emb, item_emb, jnp.zeros((_NP - n, 2 * _DH), user_emb.dtype)],
        axis=0)
    pad = _EP - edge_index.shape[1]
    srcp = jnp.concatenate(
        [edge_index[0], jnp.full((pad,), n, jnp.int32)]).reshape(-1, _G)
    dstp = jnp.concatenate(
        [edge_index[1], jnp.full((pad,), n, jnp.int32)]).reshape(-1, _G)
    zrow = jnp.zeros((_RC, _DH), jnp.float32)
    zdeg = jnp.zeros((_RPT,), jnp.float32)
    out = _propagate(srcp, dstp, x0p, zrow, zdeg)
    return (out[:nu], out[nu:n])


# speedup vs baseline: 40.0444x; 1.0097x over previous
import jax
import jax.numpy as jnp
from jax import lax
from jax.experimental import pallas as pl
from jax.experimental.pallas import tpu as pltpu
from jax.experimental.pallas import tpu_sc as plsc

_N = 50000
_NP = 50176
_DH = 16
_G = 128
_GPT = 784
_EP = 16 * _GPT * _G
_CG = 4
_NCHUNK = _GPT // _CG
_RPT = _NP // 16
_RC = 112
_NRC = _RPT // _RC
_NLAYERS = 3


def _rsqrt_newton(d):
    bits = lax.bitcast_convert_type(d, jnp.int32)
    seed = jnp.int32(0x5F3759DF) - lax.shift_right_logical(bits, 1)
    y = lax.bitcast_convert_type(seed, jnp.float32)
    for _ in range(3):
        y = y * (1.5 - 0.5 * d * y * y)
    return jnp.where(d >= 0.5, y, jnp.float32(0.0))


def _body(srcg, dstg, x0p, zrow, zdeg, out,
          y_sh, acc_sh, deg_sh,
          srcv0, dstv0, srcv1, dstv1, rows0, rows1,
          xbuf, obuf, abuf, xbuf1, obuf1, abuf1, ones_v,
          gsem, ssem0, ssem1, isem0, isem1):
    c = lax.axis_index("c")
    s = lax.axis_index("s")
    row0 = s * _RPT
    g0 = s * _GPT
    dcol = c * _DH

    def _prime_idx(gg):
        pltpu.async_copy(srcg.at[pl.ds(gg, _CG)], srcv0, isem0)
        pltpu.async_copy(dstg.at[pl.ds(gg, _CG)], dstv0, isem0)

    def _wait_idx(gg, sv, dv, isem):
        pltpu.make_async_copy(srcg.at[pl.ds(gg, _CG)], sv, isem).wait()
        pltpu.make_async_copy(dstg.at[pl.ds(gg, _CG)], dv, isem).wait()

    def _prefetch_idx(gg, sv, dv, isem):
        pltpu.async_copy(srcg.at[pl.ds(gg, _CG)], sv, isem)
        pltpu.async_copy(dstg.at[pl.ds(gg, _CG)], dv, isem)

    def _fill_ones(i, carry):
        ones_v[pl.ds(i * 16, 16)] = jnp.ones((16,), jnp.float32)
        return carry
    lax.fori_loop(0, _G // 16, _fill_ones, 0)

    def _zero_acc(k, carry):
        pltpu.sync_copy(zrow, acc_sh.at[pl.ds(row0 + k * _RC, _RC)])
        return carry

    pltpu.sync_copy(zdeg, deg_sh.at[pl.ds(row0, _RPT)])
    lax.fori_loop(0, _NRC, _zero_acc, 0)
    plsc.subcore_barrier()

    def _deg_drain(dv, ssem):
        for j in range(_CG):
            pltpu.make_async_copy(ones_v, deg_sh.at[dv.at[j]], ssem).wait()

    def _deg_phase(i, off, dv, isem, ssem, odv, oisem, ossem):
        gg = g0 + (2 * i + off) * _CG
        if off == 1:
            _deg_drain(odv, ossem)
        else:
            @pl.when(i > 0)
            def _():
                _deg_drain(odv, ossem)
        if off == 0:
            pltpu.async_copy(dstg.at[pl.ds(gg + _CG, _CG)], odv, oisem)
        else:
            @pl.when(i < _NCHUNK // 2 - 1)
            def _():
                pltpu.async_copy(dstg.at[pl.ds(gg + _CG, _CG)], odv, oisem)
        pltpu.make_async_copy(dstg.at[pl.ds(gg, _CG)], dv, isem).wait()
        for j in range(_CG):
            pltpu.async_copy(ones_v, deg_sh.at[dv.at[j]], ssem, add=True)

    pltpu.async_copy(dstg.at[pl.ds(g0, _CG)], dstv0, isem0)

    def _deg_pair(i, carry):
        _deg_phase(i, 0, dstv0, isem0, ssem0, dstv1, isem1, ssem1)
        _deg_phase(i, 1, dstv1, isem1, ssem1, dstv0, isem0, ssem0)
        return carry
    lax.fori_loop(0, _NCHUNK // 2, _deg_pair, 0)
    _deg_drain(dstv1, ssem1)
    plsc.subcore_barrier()

    def _prologue_chunk(k, carry):
        r0 = row0 + k * _RC

        pltpu.sync_copy(x0p.at[pl.ds(r0, _RC), pl.ds(dcol, _DH)], xbuf)
        pltpu.sync_copy(xbuf, out.at[pl.ds(r0, _RC), pl.ds(dcol, _DH)])
        pltpu.sync_copy(deg_sh.at[pl.ds(r0, _RC)], abuf)

        def _scale0(i, c2):
            avec = _rsqrt_newton(abuf[pl.ds(i * 16, 16)])
            for t in range(16):
                r = i * 16 + t
                xbuf[r] = xbuf[r] * avec[t]
            return c2
        lax.fori_loop(0, _RC // 16, _scale0, 0)
        pltpu.sync_copy(xbuf, y_sh.at[pl.ds(r0, _RC)])
        return carry
    lax.fori_loop(0, _NRC, _prologue_chunk, 0)
    plsc.subcore_barrier()

    def _edge_drain(rws, dv, ssem):
        for j in range(_CG):
            pltpu.make_async_copy(rws.at[j], acc_sh.at[dv.at[j]], ssem).wait()

    def _edge_phase(i, off, sv, dv, isem, ssem, rws,
                    osv, odv, oisem, ossem, orws):
        gg = g0 + (2 * i + off) * _CG
        if off == 1:
            _edge_drain(orws, odv, ossem)
        else:
            @pl.when(i > 0)
            def _():
                _edge_drain(orws, odv, ossem)
        if off == 0:
            _prefetch_idx(gg + _CG, osv, odv, oisem)
        else:
            @pl.when(i < _NCHUNK // 2 - 1)
            def _():
                _prefetch_idx(gg + _CG, osv, odv, oisem)
        _wait_idx(gg, sv, dv, isem)
        gd = [pltpu.async_copy(y_sh.at[sv.at[j]], rws.at[j], gsem)
              for j in range(_CG)]
        for j in range(_CG):
            gd[j].wait()
            pltpu.async_copy(rws.at[j], acc_sh.at[dv.at[j]], ssem, add=True)

    for layer in range(_NLAYERS):
        _prime_idx(g0)

        def _edge_pair(i, carry):
            _edge_phase(i, 0, srcv0, dstv0, isem0, ssem0, rows0,
                        srcv1, dstv1, isem1, ssem1, rows1)
            _edge_phase(i, 1, srcv1, dstv1, isem1, ssem1, rows1,
                        srcv0, dstv0, isem0, ssem0, rows0)
            return carry
        lax.fori_loop(0, _NCHUNK // 2, _edge_pair, 0)
        _edge_drain(rows1, dstv1, ssem1)
        plsc.subcore_barrier()

        last = layer == _NLAYERS - 1

        def _resc_loads(r0, xb, ab, lsem):
            pltpu.async_copy(acc_sh.at[pl.ds(r0, _RC)], xb, lsem)
            pltpu.async_copy(deg_sh.at[pl.ds(r0, _RC)], ab, lsem)

        def _resc_wait_loads(r0, xb, ab, lsem):
            pltpu.make_async_copy(acc_sh.at[pl.ds(r0, _RC)], xb, lsem).wait()
            pltpu.make_async_copy(deg_sh.at[pl.ds(r0, _RC)], ab, lsem).wait()

        def _resc_drain_stores(r0, xb, wsem, is_last):
            if not is_last:
                pltpu.make_async_copy(xb, y_sh.at[pl.ds(r0, _RC)],
                                      wsem).wait()

        def _resc_phase(k, off, xb, ob, ab, lsem, wsem,
                        oxb, oob, oab, olsem, owsem, is_last):
            r0 = row0 + (2 * k + off) * _RC
            if not is_last:
                if off == 1:
                    _resc_drain_stores(r0, oxb, owsem, is_last)
                else:
                    @pl.when(k > 0)
                    def _():
                        _resc_drain_stores(r0, oxb, owsem, is_last)
            if off == 0:
                _resc_loads(r0 + _RC, oxb, oab, olsem)
            else:
                @pl.when(k < _NRC // 2 - 1)
                def _():
                    _resc_loads(r0 + _RC, oxb, oab, olsem)
            _resc_wait_loads(r0, xb, ab, lsem)
            pltpu.sync_copy(out.at[pl.ds(r0, _RC), pl.ds(dcol, _DH)], ob)

            if not is_last:
                def _resc(i, c2):
                    avec = _rsqrt_newton(ab[pl.ds(i * 16, 16)])
                    for t in range(16):
                        r = i * 16 + t
                        xv = xb[r] * avec[t]
                        ob[r] = ob[r] + xv
                        xb[r] = xv * avec[t]
                    return c2
                lax.fori_loop(0, _RC // 16, _resc, 0)
            else:
                def _resc(i, c2):
                    avec = _rsqrt_newton(ab[pl.ds(i * 16, 16)])
                    for t in range(16):
                        r = i * 16 + t
                        xv = xb[r] * avec[t]
                        ob[r] = (ob[r] + xv) * jnp.float32(0.25)
                    return c2
                lax.fori_loop(0, _RC // 16, _resc, 0)
            pltpu.sync_copy(ob, out.at[pl.ds(r0, _RC), pl.ds(dcol, _DH)])
            if not is_last:
                pltpu.async_copy(xb, y_sh.at[pl.ds(r0, _RC)], wsem)

        _resc_loads(row0, xbuf, abuf, isem0)

        def _resc_pair(k, carry):
            _resc_phase(k, 0, xbuf, obuf, abuf, isem0, ssem0,
                        xbuf1, obuf1, abuf1, isem1, ssem1, last)
            _resc_phase(k, 1, xbuf1, obuf1, abuf1, isem1, ssem1,
                        xbuf, obuf, abuf, isem0, ssem0, last)
            return carry
        lax.fori_loop(0, _NRC // 2, _resc_pair, 0)
        _resc_drain_stores(row0, xbuf1, ssem1, last)
        if not last:
            lax.fori_loop(0, _NRC, _zero_acc, 0)
        plsc.subcore_barrier()


def _propagate(srcp, dstp, x0p, zrow, zdeg):
    mesh = plsc.VectorSubcoreMesh(core_axis_name="c", subcore_axis_name="s")
    f = pl.kernel(
        _body,
        out_type=jax.ShapeDtypeStruct((_NP, 2 * _DH), jnp.float32),
        mesh=mesh,
        scratch_types=[
            pltpu.VMEM_SHARED((_NP, _DH), jnp.float32),
            pltpu.VMEM_SHARED((_NP, _DH), jnp.float32),
            pltpu.VMEM_SHARED((_NP,), jnp.float32),
            pltpu.VMEM((_CG, _G), jnp.int32),
            pltpu.VMEM((_CG, _G), jnp.int32),
            pltpu.VMEM((_CG, _G), jnp.int32),
            pltpu.VMEM((_CG, _G), jnp.int32),
            pltpu.VMEM((_CG, _G, _DH), jnp.float32),
            pltpu.VMEM((_CG, _G, _DH), jnp.float32),
            pltpu.VMEM((_RC, _DH), jnp.float32),
            pltpu.VMEM((_RC, _DH), jnp.float32),
            pltpu.VMEM((_RC,), jnp.float32),
            pltpu.VMEM((_RC, _DH), jnp.float32),
            pltpu.VMEM((_RC, _DH), jnp.float32),
            pltpu.VMEM((_RC,), jnp.float32),
            pltpu.VMEM((_G,), jnp.float32),
            pltpu.SemaphoreType.DMA,
            pltpu.SemaphoreType.DMA,
            pltpu.SemaphoreType.DMA,
            pltpu.SemaphoreType.DMA,
            pltpu.SemaphoreType.DMA,
        ],
        compiler_params=pltpu.CompilerParams(use_tc_tiling_on_sc=False),
    )
    return f(srcp, dstp, x0p, zrow, zdeg)


def kernel(edge_index, n_users, n_items, user_emb, item_emb):
    nu = user_emb.shape[0]
    ni = item_emb.shape[0]
    n = nu + ni
    x0p = jnp.concatenate(
        [user_emb, item_emb, jnp.zeros((_NP - n, 2 * _DH), user_emb.dtype)],
        axis=0)
    pad = _EP - edge_index.shape[1]
    srcp = jnp.concatenate(
        [edge_index[0], jnp.full((pad,), n, jnp.int32)]).reshape(-1, _G)
    dstp = jnp.concatenate(
        [edge_index[1], jnp.full((pad,), n, jnp.int32)]).reshape(-1, _G)
    zrow = jnp.zeros((_RC, _DH), jnp.float32)
    zdeg = jnp.zeros((_RPT,), jnp.float32)
    out = _propagate(srcp, dstp, x0p, zrow, zdeg)
    return (out[:nu], out[nu:n])
